# Initial kernel scaffold; baseline (speedup 1.0000x reference)
import jax
import jax.numpy as jnp
from jax.experimental import pallas as pl


def kernel(xyz, points, W1, b1, gamma1, beta1, W2, b2, gamma2, beta2, W3, b3, gamma3, beta3):
    raise NotImplementedError("write your pallas kernel here")


# speedup vs baseline: 1.0043x; 1.0043x over previous
import functools

import jax
import jax.numpy as jnp
from jax.experimental import pallas as pl
from jax.experimental.pallas import tpu as pltpu

_NPOINT = 1024
_RADIUS = 0.2
_NSAMPLE = 32
_EPS = 1e-5

_B = 8
_N = 4096
_M = _B * _NPOINT
_ROWS = _M * _NSAMPLE
_TM = 256
_GRID_M = _M // _TM


def _square_distance(src, dst):
    return (jnp.sum(src ** 2, -1)[:, :, None]
            + jnp.sum(dst ** 2, -1)[:, None, :]
            - 2.0 * jnp.matmul(src, dst.transpose(0, 2, 1)))


def _index_points(points, idx):
    B = points.shape[0]
    batch = jnp.arange(B).reshape((B,) + (1,) * (idx.ndim - 1))
    return points[batch, idx]


def _farthest_point_sample(xyz, npoint):
    B, N, _ = xyz.shape

    def body(i, state):
        centroids, distance, farthest = state
        centroids = centroids.at[:, i].set(farthest)
        centroid = jnp.take_along_axis(xyz, farthest[:, None, None], axis=1)
        d = jnp.sum((xyz - centroid) ** 2, -1)
        distance = jnp.minimum(distance, d)
        farthest = jnp.argmax(distance, axis=-1).astype(jnp.int32)
        return centroids, distance, farthest

    centroids = jnp.zeros((B, npoint), dtype=jnp.int32)
    distance = jnp.full((B, N), 1e10, dtype=xyz.dtype)
    farthest = jnp.zeros((B,), dtype=jnp.int32)
    centroids, _, _ = jax.lax.fori_loop(0, npoint, body, (centroids, distance, farthest))
    return centroids


def _query_ball_point(radius, nsample, xyz, new_xyz):
    B, N, _ = xyz.shape
    S = new_xyz.shape[1]
    sqrdists = _square_distance(new_xyz, xyz)
    group_idx = jnp.broadcast_to(jnp.arange(N, dtype=jnp.int32), (B, S, N))
    group_idx = jnp.where(sqrdists > radius ** 2, N, group_idx)
    group_idx = jnp.sort(group_idx, axis=-1)[:, :, :nsample]
    group_first = jnp.broadcast_to(group_idx[:, :, :1], group_idx.shape)
    group_idx = jnp.where(group_idx == N, group_first, group_idx)
    return group_idx


def _sum_pair(x, c):
    s = jnp.sum(x, axis=0)
    q = jnp.sum(x * x, axis=0)
    pad = 128 - c
    if pad:
        s = jnp.concatenate([s, jnp.zeros((pad,), jnp.float32)])
        q = jnp.concatenate([q, jnp.zeros((pad,), jnp.float32)])
    return jnp.concatenate([s[None], q[None], jnp.zeros((6, 128), jnp.float32)], axis=0)


def _accumulate(out_ref, tile):
    @pl.when(pl.program_id(0) == 0)
    def _init():
        out_ref[...] = jnp.zeros_like(out_ref)

    out_ref[...] += tile


def _stats1_body(g_ref, w1_ref, out_ref):
    g = g_ref[...].reshape(_NSAMPLE * _TM, 8)
    x1 = jnp.dot(g, w1_ref[...], preferred_element_type=jnp.float32)
    _accumulate(out_ref, _sum_pair(x1, 64))


def _stats2_body(g_ref, w1_ref, w2_ref, out_ref):
    g = g_ref[...].reshape(_NSAMPLE * _TM, 8)
    y1 = jax.nn.relu(jnp.dot(g, w1_ref[...], preferred_element_type=jnp.float32))
    x2 = jnp.dot(y1, w2_ref[...], preferred_element_type=jnp.float32)
    _accumulate(out_ref, _sum_pair(x2, 64))


def _stats3_body(g_ref, w1_ref, w2_ref, vec_ref, w3_ref, out_ref):
    g = g_ref[...].reshape(_NSAMPLE * _TM, 8)
    shift2 = vec_ref[0, :64]
    y1 = jax.nn.relu(jnp.dot(g, w1_ref[...], preferred_element_type=jnp.float32))
    y2 = jax.nn.relu(jnp.dot(y1, w2_ref[...], preferred_element_type=jnp.float32)
                     + shift2[None, :])
    x3 = jnp.dot(y2, w3_ref[...], preferred_element_type=jnp.float32)
    _accumulate(out_ref, _sum_pair(x3, 128))


def _final_body(g_ref, w1_ref, w2_ref, vec_ref, w3_ref, out_ref):
    g = g_ref[...].reshape(_NSAMPLE * _TM, 8)
    shift2 = vec_ref[0, :64]
    shift3 = vec_ref[1, :]
    y1 = jax.nn.relu(jnp.dot(g, w1_ref[...], preferred_element_type=jnp.float32))
    y2 = jax.nn.relu(jnp.dot(y1, w2_ref[...], preferred_element_type=jnp.float32)
                     + shift2[None, :])
    y3 = jax.nn.relu(jnp.dot(y2, w3_ref[...], preferred_element_type=jnp.float32)
                     + shift3[None, :])
    out_ref[...] = jnp.max(y3.reshape(_NSAMPLE, _TM, 128), axis=0)


def _g_spec():
    return pl.BlockSpec((_NSAMPLE, _TM, 8), lambda i: (0, i, 0))


def _full_spec(shape):
    return pl.BlockSpec(shape, lambda i: (0,) * len(shape))


def _run_stats(body, g, ops):
    specs = [_g_spec()] + [_full_spec(o.shape) for o in ops]
    return pl.pallas_call(
        body,
        grid=(_GRID_M,),
        in_specs=specs,
        out_specs=pl.BlockSpec((8, 128), lambda i: (0, 0)),
        out_shape=jax.ShapeDtypeStruct((8, 128), jnp.float32),
    )(g, *ops)


def _run_final(g, ops):
    specs = [_g_spec()] + [_full_spec(o.shape) for o in ops]
    return pl.pallas_call(
        _final_body,
        grid=(_GRID_M,),
        in_specs=specs,
        out_specs=pl.BlockSpec((_TM, 128), lambda i: (i, 0)),
        out_shape=jax.ShapeDtypeStruct((_M, 128), jnp.float32),
    )(g, *ops)


def kernel(xyz, points, W1, b1, gamma1, beta1, W2, b2, gamma2, beta2,
           W3, b3, gamma3, beta3):
    xyz_t = xyz.transpose(0, 2, 1)
    pts_t = points.transpose(0, 2, 1)

    fps_idx = _farthest_point_sample(xyz_t, _NPOINT)
    new_xyz = _index_points(xyz_t, fps_idx)
    idx = _query_ball_point(_RADIUS, _NSAMPLE, xyz_t, new_xyz)

    grouped_xyz = _index_points(xyz_t, idx)
    grouped_xyz = grouped_xyz - new_xyz[:, :, None, :]
    grouped_pts = _index_points(pts_t, idx)
    ones = jnp.ones(grouped_pts.shape[:-1] + (1,), jnp.float32)
    zeros = jnp.zeros_like(ones)
    g = jnp.concatenate([grouped_xyz, grouped_pts, ones, zeros], axis=-1)
    g = g.reshape(_M, _NSAMPLE, 8).transpose(1, 0, 2)

    w1 = jnp.zeros((8, 64), jnp.float32).at[:6, :].set(W1.T).at[6, :].set(b1)
    w2 = W2.T
    w3 = W3.T
    n = float(_ROWS)

    s1 = _run_stats(_stats1_body, g, [w1])
    mean1 = s1[0, :64] / n
    var1 = s1[1, :64] / n - mean1 * mean1
    sc1 = gamma1 / jnp.sqrt(var1 + _EPS)
    w1e = (w1 * sc1[None, :]).at[6, :].add(beta1 - mean1 * sc1)

    s2 = _run_stats(_stats2_body, g, [w1e, w2])
    m2 = s2[0, :64] / n
    mean2 = m2 + b2
    var2 = s2[1, :64] / n + 2.0 * b2 * m2 + b2 * b2 - mean2 * mean2
    sc2 = gamma2 / jnp.sqrt(var2 + _EPS)
    w2e = w2 * sc2[None, :]
    shift2 = beta2 + (b2 - mean2) * sc2

    vec2 = jnp.zeros((8, 128), jnp.float32).at[0, :64].set(shift2)
    s3 = _run_stats(_stats3_body, g, [w1e, w2e, vec2, w3])
    m3 = s3[0, :] / n
    mean3 = m3 + b3
    var3 = s3[1, :] / n + 2.0 * b3 * m3 + b3 * b3 - mean3 * mean3
    sc3 = gamma3 / jnp.sqrt(var3 + _EPS)
    w3e = w3 * sc3[None, :]
    shift3 = beta3 + (b3 - mean3) * sc3

    vec23 = vec2.at[1, :].set(shift3)
    out = _run_final(g, [w1e, w2e, vec23, w3e])

    new_points = out.reshape(_B, _NPOINT, 128).transpose(0, 2, 1)
    return new_xyz.transpose(0, 2, 1), new_points


# speedup vs baseline: 1.5225x; 1.5160x over previous
import functools

import jax
import jax.numpy as jnp
from jax import lax
from jax.experimental import pallas as pl
from jax.experimental.pallas import tpu as pltpu
from jax.experimental.pallas import tpu_sc as plsc

_NPOINT = 1024
_RADIUS = 0.2
_NSAMPLE = 32
_EPS = 1e-5

_B = 8
_N = 4096
_M = _B * _NPOINT
_ROWS = _M * _NSAMPLE
_TM = 256
_GRID_M = _M // _TM


def _square_distance(src, dst):
    return (jnp.sum(src ** 2, -1)[:, :, None]
            + jnp.sum(dst ** 2, -1)[:, None, :]
            - 2.0 * jnp.matmul(src, dst.transpose(0, 2, 1)))


def _index_points(points, idx):
    B = points.shape[0]
    batch = jnp.arange(B).reshape((B,) + (1,) * (idx.ndim - 1))
    return points[batch, idx]


def _farthest_point_sample(xyz, npoint):
    B, N, _ = xyz.shape

    def body(i, state):
        centroids, distance, farthest = state
        centroids = centroids.at[:, i].set(farthest)
        centroid = jnp.take_along_axis(xyz, farthest[:, None, None], axis=1)
        d = jnp.sum((xyz - centroid) ** 2, -1)
        distance = jnp.minimum(distance, d)
        farthest = jnp.argmax(distance, axis=-1).astype(jnp.int32)
        return centroids, distance, farthest

    centroids = jnp.zeros((B, npoint), dtype=jnp.int32)
    distance = jnp.full((B, N), 1e10, dtype=xyz.dtype)
    farthest = jnp.zeros((B,), dtype=jnp.int32)
    centroids, _, _ = jax.lax.fori_loop(0, npoint, body, (centroids, distance, farthest))
    return centroids


def _query_ball_point(radius, nsample, xyz, new_xyz):
    B, N, _ = xyz.shape
    S = new_xyz.shape[1]
    sqrdists = _square_distance(new_xyz, xyz)
    group_idx = jnp.broadcast_to(jnp.arange(N, dtype=jnp.int32), (B, S, N))
    group_idx = jnp.where(sqrdists > radius ** 2, N, group_idx)
    group_idx = jnp.sort(group_idx, axis=-1)[:, :, :nsample]
    group_first = jnp.broadcast_to(group_idx[:, :, :1], group_idx.shape)
    group_idx = jnp.where(group_idx == N, group_first, group_idx)
    return group_idx


_NW = 32
_RPW = _ROWS // _NW
_CHUNK = 128
_NCHUNK = _RPW // _CHUNK


def _sc_gather_body(table_ref, idx_ref, out_ref, idxv, buf0, buf1, sem0, sem1):
    wid = lax.axis_index("s") * 2 + lax.axis_index("c")
    pltpu.sync_copy(idx_ref.at[wid], idxv)
    base = wid * _RPW

    def start(j, buf, sem):
        return pltpu.async_copy(table_ref.at[idxv.at[j]], buf, sem)

    start(0, buf0, sem0)

    def body(j, carry):
        @pl.when(j % 2 == 0)
        def _even():
            @pl.when(j + 1 < _NCHUNK)
            def _s():
                start(j + 1, buf1, sem1)
            pltpu.make_async_copy(table_ref.at[idxv.at[j]], buf0, sem0).wait()
            pltpu.sync_copy(buf0, out_ref.at[pl.ds(base + j * _CHUNK, _CHUNK)])

        @pl.when(j % 2 == 1)
        def _odd():
            @pl.when(j + 1 < _NCHUNK)
            def _s():
                start(j + 1, buf0, sem0)
            pltpu.make_async_copy(table_ref.at[idxv.at[j]], buf1, sem1).wait()
            pltpu.sync_copy(buf1, out_ref.at[pl.ds(base + j * _CHUNK, _CHUNK)])

        return carry

    lax.fori_loop(0, _NCHUNK, body, 0)


def _sc_gather(table16, idxw):
    return pl.kernel(
        _sc_gather_body,
        out_type=jax.ShapeDtypeStruct((_ROWS, 16), jnp.float32),
        mesh=plsc.VectorSubcoreMesh(core_axis_name="c", subcore_axis_name="s"),
        compiler_params=pltpu.CompilerParams(use_tc_tiling_on_sc=False),
        scratch_types=[
            pltpu.VMEM((_NCHUNK, _CHUNK), jnp.int32),
            pltpu.VMEM((_CHUNK, 16), jnp.float32),
            pltpu.VMEM((_CHUNK, 16), jnp.float32),
            pltpu.SemaphoreType.DMA,
            pltpu.SemaphoreType.DMA,
        ],
    )(table16, idxw)


def _sum_pair(x, c):
    s = jnp.sum(x, axis=0)
    q = jnp.sum(x * x, axis=0)
    pad = 128 - c
    if pad:
        s = jnp.concatenate([s, jnp.zeros((pad,), jnp.float32)])
        q = jnp.concatenate([q, jnp.zeros((pad,), jnp.float32)])
    return jnp.concatenate([s[None], q[None], jnp.zeros((6, 128), jnp.float32)], axis=0)


def _accumulate(out_ref, tile):
    @pl.when(pl.program_id(0) == 0)
    def _init():
        out_ref[...] = jnp.zeros_like(out_ref)

    out_ref[...] += tile


def _stats1_body(g_ref, nx_ref, w1_ref, out_ref):
    g = (g_ref[...] - nx_ref[...][None]).reshape(_NSAMPLE * _TM, 16)
    x1 = jnp.dot(g, w1_ref[...], preferred_element_type=jnp.float32)
    _accumulate(out_ref, _sum_pair(x1, 64))


def _stats2_body(g_ref, nx_ref, w1_ref, w2_ref, out_ref):
    g = (g_ref[...] - nx_ref[...][None]).reshape(_NSAMPLE * _TM, 16)
    y1 = jax.nn.relu(jnp.dot(g, w1_ref[...], preferred_element_type=jnp.float32))
    x2 = jnp.dot(y1, w2_ref[...], preferred_element_type=jnp.float32)
    _accumulate(out_ref, _sum_pair(x2, 64))


def _stats3_body(g_ref, nx_ref, w1_ref, w2_ref, vec_ref, w3_ref, out_ref):
    g = (g_ref[...] - nx_ref[...][None]).reshape(_NSAMPLE * _TM, 16)
    shift2 = vec_ref[0, :64]
    y1 = jax.nn.relu(jnp.dot(g, w1_ref[...], preferred_element_type=jnp.float32))
    y2 = jax.nn.relu(jnp.dot(y1, w2_ref[...], preferred_element_type=jnp.float32)
                     + shift2[None, :])
    x3 = jnp.dot(y2, w3_ref[...], preferred_element_type=jnp.float32)
    _accumulate(out_ref, _sum_pair(x3, 128))


def _final_body(g_ref, nx_ref, w1_ref, w2_ref, vec_ref, w3_ref, out_ref):
    g = (g_ref[...] - nx_ref[...][None]).reshape(_NSAMPLE * _TM, 16)
    shift2 = vec_ref[0, :64]
    shift3 = vec_ref[1, :]
    y1 = jax.nn.relu(jnp.dot(g, w1_ref[...], preferred_element_type=jnp.float32))
    y2 = jax.nn.relu(jnp.dot(y1, w2_ref[...], preferred_element_type=jnp.float32)
                     + shift2[None, :])
    y3 = jax.nn.relu(jnp.dot(y2, w3_ref[...], preferred_element_type=jnp.float32)
                     + shift3[None, :])
    out_ref[...] = jnp.max(y3.reshape(_NSAMPLE, _TM, 128), axis=0)


def _g_spec():
    return pl.BlockSpec((_NSAMPLE, _TM, 16), lambda i: (0, i, 0))


def _nx_spec():
    return pl.BlockSpec((_TM, 16), lambda i: (i, 0))


def _full_spec(shape):
    return pl.BlockSpec(shape, lambda i: (0,) * len(shape))


def _run_stats(body, g, nx, ops):
    specs = [_g_spec(), _nx_spec()] + [_full_spec(o.shape) for o in ops]
    return pl.pallas_call(
        body,
        grid=(_GRID_M,),
        in_specs=specs,
        out_specs=pl.BlockSpec((8, 128), lambda i: (0, 0)),
        out_shape=jax.ShapeDtypeStruct((8, 128), jnp.float32),
    )(g, nx, *ops)


def _run_final(g, nx, ops):
    specs = [_g_spec(), _nx_spec()] + [_full_spec(o.shape) for o in ops]
    return pl.pallas_call(
        _final_body,
        grid=(_GRID_M,),
        in_specs=specs,
        out_specs=pl.BlockSpec((_TM, 128), lambda i: (i, 0)),
        out_shape=jax.ShapeDtypeStruct((_M, 128), jnp.float32),
    )(g, nx, *ops)


def kernel(xyz, points, W1, b1, gamma1, beta1, W2, b2, gamma2, beta2,
           W3, b3, gamma3, beta3):
    xyz_t = xyz.transpose(0, 2, 1)
    pts_t = points.transpose(0, 2, 1)

    fps_idx = _farthest_point_sample(xyz_t, _NPOINT)
    new_xyz = _index_points(xyz_t, fps_idx)
    idx = _query_ball_point(_RADIUS, _NSAMPLE, xyz_t, new_xyz)

    table16 = jnp.concatenate(
        [xyz_t, pts_t,
         jnp.ones((_B, _N, 1), jnp.float32),
         jnp.zeros((_B, _N, 9), jnp.float32)], axis=-1).reshape(_B * _N, 16)
    offs = (jnp.arange(_B, dtype=jnp.int32) * _N)[:, None, None]
    flat_idx = (idx + offs).reshape(_M, _NSAMPLE).T
    idxw = flat_idx.reshape(_NW, _NCHUNK, _CHUNK)
    g = _sc_gather(table16, idxw).reshape(_NSAMPLE, _M, 16)

    nxpad = jnp.concatenate(
        [new_xyz.reshape(_M, 3), jnp.zeros((_M, 13), jnp.float32)], axis=-1)

    w1 = jnp.zeros((16, 64), jnp.float32).at[:6, :].set(W1.T).at[6, :].set(b1)
    w2 = W2.T
    w3 = W3.T
    n = float(_ROWS)

    s1 = _run_stats(_stats1_body, g, nxpad, [w1])
    mean1 = s1[0, :64] / n
    var1 = s1[1, :64] / n - mean1 * mean1
    sc1 = gamma1 / jnp.sqrt(var1 + _EPS)
    w1e = (w1 * sc1[None, :]).at[6, :].add(beta1 - mean1 * sc1)

    s2 = _run_stats(_stats2_body, g, nxpad, [w1e, w2])
    m2 = s2[0, :64] / n
    mean2 = m2 + b2
    var2 = s2[1, :64] / n + 2.0 * b2 * m2 + b2 * b2 - mean2 * mean2
    sc2 = gamma2 / jnp.sqrt(var2 + _EPS)
    w2e = w2 * sc2[None, :]
    shift2 = beta2 + (b2 - mean2) * sc2

    vec2 = jnp.zeros((8, 128), jnp.float32).at[0, :64].set(shift2)
    s3 = _run_stats(_stats3_body, g, nxpad, [w1e, w2e, vec2, w3])
    m3 = s3[0, :] / n
    mean3 = m3 + b3
    var3 = s3[1, :] / n + 2.0 * b3 * m3 + b3 * b3 - mean3 * mean3
    sc3 = gamma3 / jnp.sqrt(var3 + _EPS)
    w3e = w3 * sc3[None, :]
    shift3 = beta3 + (b3 - mean3) * sc3

    vec23 = vec2.at[1, :].set(shift3)
    out = _run_final(g, nxpad, [w1e, w2e, vec23, w3e])

    new_points = out.reshape(_B, _NPOINT, 128).transpose(0, 2, 1)
    return new_xyz.transpose(0, 2, 1), new_points


# speedup vs baseline: 3.8630x; 2.5372x over previous
import functools

import jax
import jax.numpy as jnp
from jax import lax
from jax.experimental import pallas as pl
from jax.experimental.pallas import tpu as pltpu
from jax.experimental.pallas import tpu_sc as plsc

_NPOINT = 1024
_RADIUS = 0.2
_NSAMPLE = 32
_EPS = 1e-5

_B = 8
_N = 4096
_M = _B * _NPOINT
_ROWS = _M * _NSAMPLE
_TM = 256
_GRID_M = _M // _TM


def _square_distance(src, dst):
    return (jnp.sum(src ** 2, -1)[:, :, None]
            + jnp.sum(dst ** 2, -1)[:, None, :]
            - 2.0 * jnp.matmul(src, dst.transpose(0, 2, 1)))


def _index_points(points, idx):
    B = points.shape[0]
    batch = jnp.arange(B).reshape((B,) + (1,) * (idx.ndim - 1))
    return points[batch, idx]


def _farthest_point_sample(xyz, npoint):
    B, N, _ = xyz.shape

    def body(i, state):
        centroids, distance, farthest = state
        centroids = centroids.at[:, i].set(farthest)
        centroid = jnp.take_along_axis(xyz, farthest[:, None, None], axis=1)
        d = jnp.sum((xyz - centroid) ** 2, -1)
        distance = jnp.minimum(distance, d)
        farthest = jnp.argmax(distance, axis=-1).astype(jnp.int32)
        return centroids, distance, farthest

    centroids = jnp.zeros((B, npoint), dtype=jnp.int32)
    distance = jnp.full((B, N), 1e10, dtype=xyz.dtype)
    farthest = jnp.zeros((B,), dtype=jnp.int32)
    centroids, _, _ = jax.lax.fori_loop(0, npoint, body, (centroids, distance, farthest))
    return centroids


def _query_ball_point(radius, nsample, xyz, new_xyz):
    B, N, _ = xyz.shape
    S = new_xyz.shape[1]
    sqrdists = _square_distance(new_xyz, xyz)
    group_idx = jnp.broadcast_to(jnp.arange(N, dtype=jnp.int32), (B, S, N))
    group_idx = jnp.where(sqrdists > radius ** 2, N, group_idx)
    group_idx = jnp.sort(group_idx, axis=-1)[:, :, :nsample]
    group_first = jnp.broadcast_to(group_idx[:, :, :1], group_idx.shape)
    group_idx = jnp.where(group_idx == N, group_first, group_idx)
    return group_idx


def _fps_body(xyz_ref, cx_ref, cy_ref, cz_ref):
    xr = xyz_ref[0]
    yr = xyz_ref[1]
    zr = xyz_ref[2]
    iota_n = lax.broadcasted_iota(jnp.int32, (_B, _N), 1)
    iota_s = lax.broadcasted_iota(jnp.int32, (_B, _NPOINT), 1)

    def body(i, st):
        dist, farv, cxa, cya, cza = st
        sel = iota_n == farv
        cx = jnp.sum(jnp.where(sel, xr, 0.0), axis=1, keepdims=True)
        cy = jnp.sum(jnp.where(sel, yr, 0.0), axis=1, keepdims=True)
        cz = jnp.sum(jnp.where(sel, zr, 0.0), axis=1, keepdims=True)
        col = iota_s == i
        cxa = jnp.where(col, cx, cxa)
        cya = jnp.where(col, cy, cya)
        cza = jnp.where(col, cz, cza)
        dx = xr - cx
        dy = yr - cy
        dz = zr - cz
        d = dx * dx + dy * dy + dz * dz
        dist = jnp.minimum(dist, d)
        maxv = jnp.max(dist, axis=1, keepdims=True)
        masked = jnp.where(dist == maxv, iota_n, _N)
        farv = jnp.min(masked, axis=1, keepdims=True)
        return dist, farv, cxa, cya, cza

    init = (jnp.full((_B, _N), 1e10, jnp.float32),
            jnp.zeros((_B, 1), jnp.int32),
            jnp.zeros((_B, _NPOINT), jnp.float32),
            jnp.zeros((_B, _NPOINT), jnp.float32),
            jnp.zeros((_B, _NPOINT), jnp.float32))
    _, _, cxa, cya, cza = lax.fori_loop(0, _NPOINT, body, init)
    cx_ref[...] = cxa
    cy_ref[...] = cya
    cz_ref[...] = cza


def _fps_pallas(xyz):
    xyz3 = xyz.transpose(1, 0, 2)
    shp = jax.ShapeDtypeStruct((_B, _NPOINT), jnp.float32)
    return pl.pallas_call(
        _fps_body,
        out_shape=[shp, shp, shp],
    )(xyz3)


_NW = 32
_RPW = _ROWS // _NW
_CHUNK = 128
_NCHUNK = _RPW // _CHUNK


def _sc_gather_body(table_ref, idx_ref, out_ref, idxv, buf0, buf1, sem0, sem1):
    wid = lax.axis_index("s") * 2 + lax.axis_index("c")
    pltpu.sync_copy(idx_ref.at[wid], idxv)
    base = wid * _RPW

    def start(j, buf, sem):
        return pltpu.async_copy(table_ref.at[idxv.at[j]], buf, sem)

    start(0, buf0, sem0)

    def body(j, carry):
        @pl.when(j % 2 == 0)
        def _even():
            @pl.when(j + 1 < _NCHUNK)
            def _s():
                start(j + 1, buf1, sem1)
            pltpu.make_async_copy(table_ref.at[idxv.at[j]], buf0, sem0).wait()
            pltpu.sync_copy(buf0, out_ref.at[pl.ds(base + j * _CHUNK, _CHUNK)])

        @pl.when(j % 2 == 1)
        def _odd():
            @pl.when(j + 1 < _NCHUNK)
            def _s():
                start(j + 1, buf0, sem0)
            pltpu.make_async_copy(table_ref.at[idxv.at[j]], buf1, sem1).wait()
            pltpu.sync_copy(buf1, out_ref.at[pl.ds(base + j * _CHUNK, _CHUNK)])

        return carry

    lax.fori_loop(0, _NCHUNK, body, 0)


def _sc_gather(table16, idxw):
    return pl.kernel(
        _sc_gather_body,
        out_type=jax.ShapeDtypeStruct((_ROWS, 16), jnp.float32),
        mesh=plsc.VectorSubcoreMesh(core_axis_name="c", subcore_axis_name="s"),
        compiler_params=pltpu.CompilerParams(use_tc_tiling_on_sc=False),
        scratch_types=[
            pltpu.VMEM((_NCHUNK, _CHUNK), jnp.int32),
            pltpu.VMEM((_CHUNK, 16), jnp.float32),
            pltpu.VMEM((_CHUNK, 16), jnp.float32),
            pltpu.SemaphoreType.DMA,
            pltpu.SemaphoreType.DMA,
        ],
    )(table16, idxw)


def _sum_pair(x, c):
    s = jnp.sum(x, axis=0)
    q = jnp.sum(x * x, axis=0)
    pad = 128 - c
    if pad:
        s = jnp.concatenate([s, jnp.zeros((pad,), jnp.float32)])
        q = jnp.concatenate([q, jnp.zeros((pad,), jnp.float32)])
    return jnp.concatenate([s[None], q[None], jnp.zeros((6, 128), jnp.float32)], axis=0)


def _accumulate(out_ref, tile):
    @pl.when(pl.program_id(0) == 0)
    def _init():
        out_ref[...] = jnp.zeros_like(out_ref)

    out_ref[...] += tile


def _stats1_body(g_ref, nx_ref, w1_ref, out_ref):
    g = (g_ref[...] - nx_ref[...][None]).reshape(_NSAMPLE * _TM, 16)
    x1 = jnp.dot(g, w1_ref[...], preferred_element_type=jnp.float32)
    _accumulate(out_ref, _sum_pair(x1, 64))


def _stats2_body(g_ref, nx_ref, w1_ref, w2_ref, out_ref):
    g = (g_ref[...] - nx_ref[...][None]).reshape(_NSAMPLE * _TM, 16)
    y1 = jax.nn.relu(jnp.dot(g, w1_ref[...], preferred_element_type=jnp.float32))
    x2 = jnp.dot(y1, w2_ref[...], preferred_element_type=jnp.float32)
    _accumulate(out_ref, _sum_pair(x2, 64))


def _stats3_body(g_ref, nx_ref, w1_ref, w2_ref, vec_ref, w3_ref, out_ref):
    g = (g_ref[...] - nx_ref[...][None]).reshape(_NSAMPLE * _TM, 16)
    shift2 = vec_ref[0, :64]
    y1 = jax.nn.relu(jnp.dot(g, w1_ref[...], preferred_element_type=jnp.float32))
    y2 = jax.nn.relu(jnp.dot(y1, w2_ref[...], preferred_element_type=jnp.float32)
                     + shift2[None, :])
    x3 = jnp.dot(y2, w3_ref[...], preferred_element_type=jnp.float32)
    _accumulate(out_ref, _sum_pair(x3, 128))


def _final_body(g_ref, nx_ref, w1_ref, w2_ref, vec_ref, w3_ref, out_ref):
    g = (g_ref[...] - nx_ref[...][None]).reshape(_NSAMPLE * _TM, 16)
    shift2 = vec_ref[0, :64]
    shift3 = vec_ref[1, :]
    y1 = jax.nn.relu(jnp.dot(g, w1_ref[...], preferred_element_type=jnp.float32))
    y2 = jax.nn.relu(jnp.dot(y1, w2_ref[...], preferred_element_type=jnp.float32)
                     + shift2[None, :])
    y3 = jax.nn.relu(jnp.dot(y2, w3_ref[...], preferred_element_type=jnp.float32)
                     + shift3[None, :])
    out_ref[...] = jnp.max(y3.reshape(_NSAMPLE, _TM, 128), axis=0)


def _g_spec():
    return pl.BlockSpec((_NSAMPLE, _TM, 16), lambda i: (0, i, 0))


def _nx_spec():
    return pl.BlockSpec((_TM, 16), lambda i: (i, 0))


def _full_spec(shape):
    return pl.BlockSpec(shape, lambda i: (0,) * len(shape))


def _run_stats(body, g, nx, ops):
    specs = [_g_spec(), _nx_spec()] + [_full_spec(o.shape) for o in ops]
    return pl.pallas_call(
        body,
        grid=(_GRID_M,),
        in_specs=specs,
        out_specs=pl.BlockSpec((8, 128), lambda i: (0, 0)),
        out_shape=jax.ShapeDtypeStruct((8, 128), jnp.float32),
    )(g, nx, *ops)


def _run_final(g, nx, ops):
    specs = [_g_spec(), _nx_spec()] + [_full_spec(o.shape) for o in ops]
    return pl.pallas_call(
        _final_body,
        grid=(_GRID_M,),
        in_specs=specs,
        out_specs=pl.BlockSpec((_TM, 128), lambda i: (i, 0)),
        out_shape=jax.ShapeDtypeStruct((_M, 128), jnp.float32),
    )(g, nx, *ops)


def kernel(xyz, points, W1, b1, gamma1, beta1, W2, b2, gamma2, beta2,
           W3, b3, gamma3, beta3):
    xyz_t = xyz.transpose(0, 2, 1)
    pts_t = points.transpose(0, 2, 1)

    cx, cy, cz = _fps_pallas(xyz)
    new_xyz = jnp.stack([cx, cy, cz], axis=-1)
    idx = _query_ball_point(_RADIUS, _NSAMPLE, xyz_t, new_xyz)

    table16 = jnp.concatenate(
        [xyz_t, pts_t,
         jnp.ones((_B, _N, 1), jnp.float32),
         jnp.zeros((_B, _N, 9), jnp.float32)], axis=-1).reshape(_B * _N, 16)
    offs = (jnp.arange(_B, dtype=jnp.int32) * _N)[:, None, None]
    flat_idx = (idx + offs).reshape(_M, _NSAMPLE).T
    idxw = flat_idx.reshape(_NW, _NCHUNK, _CHUNK)
    g = _sc_gather(table16, idxw).reshape(_NSAMPLE, _M, 16)

    nxpad = jnp.concatenate(
        [new_xyz.reshape(_M, 3), jnp.zeros((_M, 13), jnp.float32)], axis=-1)

    w1 = jnp.zeros((16, 64), jnp.float32).at[:6, :].set(W1.T).at[6, :].set(b1)
    w2 = W2.T
    w3 = W3.T
    n = float(_ROWS)

    s1 = _run_stats(_stats1_body, g, nxpad, [w1])
    mean1 = s1[0, :64] / n
    var1 = s1[1, :64] / n - mean1 * mean1
    sc1 = gamma1 / jnp.sqrt(var1 + _EPS)
    w1e = (w1 * sc1[None, :]).at[6, :].add(beta1 - mean1 * sc1)

    s2 = _run_stats(_stats2_body, g, nxpad, [w1e, w2])
    m2 = s2[0, :64] / n
    mean2 = m2 + b2
    var2 = s2[1, :64] / n + 2.0 * b2 * m2 + b2 * b2 - mean2 * mean2
    sc2 = gamma2 / jnp.sqrt(var2 + _EPS)
    w2e = w2 * sc2[None, :]
    shift2 = beta2 + (b2 - mean2) * sc2

    vec2 = jnp.zeros((8, 128), jnp.float32).at[0, :64].set(shift2)
    s3 = _run_stats(_stats3_body, g, nxpad, [w1e, w2e, vec2, w3])
    m3 = s3[0, :] / n
    mean3 = m3 + b3
    var3 = s3[1, :] / n + 2.0 * b3 * m3 + b3 * b3 - mean3 * mean3
    sc3 = gamma3 / jnp.sqrt(var3 + _EPS)
    w3e = w3 * sc3[None, :]
    shift3 = beta3 + (b3 - mean3) * sc3

    vec23 = vec2.at[1, :].set(shift3)
    out = _run_final(g, nxpad, [w1e, w2e, vec23, w3e])

    new_points = out.reshape(_B, _NPOINT, 128).transpose(0, 2, 1)
    return new_xyz.transpose(0, 2, 1), new_points


# speedup vs baseline: 15.6349x; 4.0473x over previous
import functools

import jax
import jax.numpy as jnp
from jax import lax
from jax.experimental import pallas as pl
from jax.experimental.pallas import tpu as pltpu
from jax.experimental.pallas import tpu_sc as plsc

_NPOINT = 1024
_RADIUS = 0.2
_NSAMPLE = 32
_EPS = 1e-5

_B = 8
_N = 4096
_M = _B * _NPOINT
_ROWS = _M * _NSAMPLE
_TM = 256
_GRID_M = _M // _TM


def _fps_body(xyz_ref, cx_ref, cy_ref, cz_ref):
    xr = xyz_ref[0]
    yr = xyz_ref[1]
    zr = xyz_ref[2]
    iota_n = lax.broadcasted_iota(jnp.int32, (_B, _N), 1)
    iota_s = lax.broadcasted_iota(jnp.int32, (_B, _NPOINT), 1)

    def body(i, st):
        dist, farv, cxa, cya, cza = st
        sel = iota_n == farv
        cx = jnp.sum(jnp.where(sel, xr, 0.0), axis=1, keepdims=True)
        cy = jnp.sum(jnp.where(sel, yr, 0.0), axis=1, keepdims=True)
        cz = jnp.sum(jnp.where(sel, zr, 0.0), axis=1, keepdims=True)
        col = iota_s == i
        cxa = jnp.where(col, cx, cxa)
        cya = jnp.where(col, cy, cya)
        cza = jnp.where(col, cz, cza)
        dx = xr - cx
        dy = yr - cy
        dz = zr - cz
        d = dx * dx + dy * dy + dz * dz
        dist = jnp.minimum(dist, d)
        maxv = jnp.max(dist, axis=1, keepdims=True)
        masked = jnp.where(dist == maxv, iota_n, _N)
        farv = jnp.min(masked, axis=1, keepdims=True)
        return dist, farv, cxa, cya, cza

    init = (jnp.full((_B, _N), 1e10, jnp.float32),
            jnp.zeros((_B, 1), jnp.int32),
            jnp.zeros((_B, _NPOINT), jnp.float32),
            jnp.zeros((_B, _NPOINT), jnp.float32),
            jnp.zeros((_B, _NPOINT), jnp.float32))
    _, _, cxa, cya, cza = lax.fori_loop(0, _NPOINT, body, init)
    cx_ref[...] = cxa
    cy_ref[...] = cya
    cz_ref[...] = cza


def _fps_pallas(xyz):
    xyz3 = xyz.transpose(1, 0, 2)
    shp = jax.ShapeDtypeStruct((_B, _NPOINT), jnp.float32)
    return pl.pallas_call(
        _fps_body,
        out_shape=[shp, shp, shp],
    )(xyz3)


_TS = 256


def _bq_select_body(xyz_ref, nx_ref, out_ref):
    xp = xyz_ref[0]
    d2 = jnp.sum(xp * xp, axis=0, keepdims=True)
    nx = nx_ref[...]
    s2 = jnp.sum(nx * nx, axis=1, keepdims=True)
    dots = jnp.dot(nx, xp, preferred_element_type=jnp.float32)
    d = s2 + d2 - 2.0 * dots
    iota_n = lax.broadcasted_iota(jnp.int32, (_TS, _N), 1)
    a = jnp.where(d <= _RADIUS ** 2, iota_n, iota_n + _N)
    cols = []
    for _ in range(_NSAMPLE):
        m = jnp.min(a, axis=1, keepdims=True)
        cols.append(m)
        a = jnp.where(a == m, a + 4 * _N, a)
    idx = jnp.concatenate(cols, axis=1)
    first = idx[:, :1]
    idx = jnp.where(idx >= _N, jnp.broadcast_to(first, idx.shape), idx)
    boff = (pl.program_id(0) // (_NPOINT // _TS)) * _N
    out_ref[...] = idx + boff


def _bq_select(xyz8, nx8):
    return pl.pallas_call(
        _bq_select_body,
        grid=(_M // _TS,),
        in_specs=[
            pl.BlockSpec((1, 8, _N), lambda i: (i // (_NPOINT // _TS), 0, 0)),
            pl.BlockSpec((_TS, 8), lambda i: (i, 0)),
        ],
        out_specs=pl.BlockSpec((_TS, _NSAMPLE), lambda i: (i, 0)),
        out_shape=jax.ShapeDtypeStruct((_M, _NSAMPLE), jnp.int32),
    )(xyz8, nx8)


_NW = 32
_RPW = _ROWS // _NW
_CHUNK = 128
_NCHUNK = _RPW // _CHUNK


def _sc_gather_body(table_ref, idx_ref, out_ref, idxv, buf0, buf1, sem0, sem1):
    wid = lax.axis_index("s") * 2 + lax.axis_index("c")
    pltpu.sync_copy(idx_ref.at[wid], idxv)
    base = wid * _RPW

    def start(j, buf, sem):
        return pltpu.async_copy(table_ref.at[idxv.at[j]], buf, sem)

    start(0, buf0, sem0)

    def body(j, carry):
        @pl.when(j % 2 == 0)
        def _even():
            @pl.when(j + 1 < _NCHUNK)
            def _s():
                start(j + 1, buf1, sem1)
            pltpu.make_async_copy(table_ref.at[idxv.at[j]], buf0, sem0).wait()
            pltpu.sync_copy(buf0, out_ref.at[pl.ds(base + j * _CHUNK, _CHUNK)])

        @pl.when(j % 2 == 1)
        def _odd():
            @pl.when(j + 1 < _NCHUNK)
            def _s():
                start(j + 1, buf0, sem0)
            pltpu.make_async_copy(table_ref.at[idxv.at[j]], buf1, sem1).wait()
            pltpu.sync_copy(buf1, out_ref.at[pl.ds(base + j * _CHUNK, _CHUNK)])

        return carry

    lax.fori_loop(0, _NCHUNK, body, 0)


def _sc_gather(table16, idxw):
    return pl.kernel(
        _sc_gather_body,
        out_type=jax.ShapeDtypeStruct((_ROWS, 16), jnp.float32),
        mesh=plsc.VectorSubcoreMesh(core_axis_name="c", subcore_axis_name="s"),
        compiler_params=pltpu.CompilerParams(use_tc_tiling_on_sc=False),
        scratch_types=[
            pltpu.VMEM((_NCHUNK, _CHUNK), jnp.int32),
            pltpu.VMEM((_CHUNK, 16), jnp.float32),
            pltpu.VMEM((_CHUNK, 16), jnp.float32),
            pltpu.SemaphoreType.DMA,
            pltpu.SemaphoreType.DMA,
        ],
    )(table16, idxw)


def _sum_pair(x, c):
    s = jnp.sum(x, axis=0)
    q = jnp.sum(x * x, axis=0)
    pad = 128 - c
    if pad:
        s = jnp.concatenate([s, jnp.zeros((pad,), jnp.float32)])
        q = jnp.concatenate([q, jnp.zeros((pad,), jnp.float32)])
    return jnp.concatenate([s[None], q[None], jnp.zeros((6, 128), jnp.float32)], axis=0)


def _accumulate(out_ref, tile):
    @pl.when(pl.program_id(0) == 0)
    def _init():
        out_ref[...] = jnp.zeros_like(out_ref)

    out_ref[...] += tile


def _stats1_body(g_ref, nx_ref, w1_ref, out_ref):
    g = (g_ref[...] - nx_ref[...][None]).reshape(_NSAMPLE * _TM, 16)
    x1 = jnp.dot(g, w1_ref[...], preferred_element_type=jnp.float32)
    _accumulate(out_ref, _sum_pair(x1, 64))


def _stats2_body(g_ref, nx_ref, w1_ref, w2_ref, out_ref):
    g = (g_ref[...] - nx_ref[...][None]).reshape(_NSAMPLE * _TM, 16)
    y1 = jax.nn.relu(jnp.dot(g, w1_ref[...], preferred_element_type=jnp.float32))
    x2 = jnp.dot(y1, w2_ref[...], preferred_element_type=jnp.float32)
    _accumulate(out_ref, _sum_pair(x2, 64))


def _stats3_body(g_ref, nx_ref, w1_ref, w2_ref, vec_ref, w3_ref, out_ref):
    g = (g_ref[...] - nx_ref[...][None]).reshape(_NSAMPLE * _TM, 16)
    shift2 = vec_ref[0, :64]
    y1 = jax.nn.relu(jnp.dot(g, w1_ref[...], preferred_element_type=jnp.float32))
    y2 = jax.nn.relu(jnp.dot(y1, w2_ref[...], preferred_element_type=jnp.float32)
                     + shift2[None, :])
    x3 = jnp.dot(y2, w3_ref[...], preferred_element_type=jnp.float32)
    _accumulate(out_ref, _sum_pair(x3, 128))


def _final_body(g_ref, nx_ref, w1_ref, w2_ref, vec_ref, w3_ref, out_ref):
    g = (g_ref[...] - nx_ref[...][None]).reshape(_NSAMPLE * _TM, 16)
    shift2 = vec_ref[0, :64]
    shift3 = vec_ref[1, :]
    y1 = jax.nn.relu(jnp.dot(g, w1_ref[...], preferred_element_type=jnp.float32))
    y2 = jax.nn.relu(jnp.dot(y1, w2_ref[...], preferred_element_type=jnp.float32)
                     + shift2[None, :])
    y3 = jax.nn.relu(jnp.dot(y2, w3_ref[...], preferred_element_type=jnp.float32)
                     + shift3[None, :])
    out_ref[...] = jnp.max(y3.reshape(_NSAMPLE, _TM, 128), axis=0)


def _g_spec():
    return pl.BlockSpec((_NSAMPLE, _TM, 16), lambda i: (0, i, 0))


def _nx_spec():
    return pl.BlockSpec((_TM, 16), lambda i: (i, 0))


def _full_spec(shape):
    return pl.BlockSpec(shape, lambda i: (0,) * len(shape))


def _run_stats(body, g, nx, ops):
    specs = [_g_spec(), _nx_spec()] + [_full_spec(o.shape) for o in ops]
    return pl.pallas_call(
        body,
        grid=(_GRID_M,),
        in_specs=specs,
        out_specs=pl.BlockSpec((8, 128), lambda i: (0, 0)),
        out_shape=jax.ShapeDtypeStruct((8, 128), jnp.float32),
    )(g, nx, *ops)


def _run_final(g, nx, ops):
    specs = [_g_spec(), _nx_spec()] + [_full_spec(o.shape) for o in ops]
    return pl.pallas_call(
        _final_body,
        grid=(_GRID_M,),
        in_specs=specs,
        out_specs=pl.BlockSpec((_TM, 128), lambda i: (i, 0)),
        out_shape=jax.ShapeDtypeStruct((_M, 128), jnp.float32),
    )(g, nx, *ops)


def kernel(xyz, points, W1, b1, gamma1, beta1, W2, b2, gamma2, beta2,
           W3, b3, gamma3, beta3):
    xyz_t = xyz.transpose(0, 2, 1)
    pts_t = points.transpose(0, 2, 1)

    cx, cy, cz = _fps_pallas(xyz)
    new_xyz = jnp.stack([cx, cy, cz], axis=-1)

    xyz8 = jnp.concatenate([xyz, jnp.zeros((_B, 5, _N), jnp.float32)], axis=1)
    nx8 = jnp.concatenate(
        [new_xyz.reshape(_M, 3), jnp.zeros((_M, 5), jnp.float32)], axis=-1)
    idx = _bq_select(xyz8, nx8)

    table16 = jnp.concatenate(
        [xyz_t, pts_t,
         jnp.ones((_B, _N, 1), jnp.float32),
         jnp.zeros((_B, _N, 9), jnp.float32)], axis=-1).reshape(_B * _N, 16)
    idxw = idx.T.reshape(_NW, _NCHUNK, _CHUNK)
    g = _sc_gather(table16, idxw).reshape(_NSAMPLE, _M, 16)

    nxpad = jnp.concatenate(
        [new_xyz.reshape(_M, 3), jnp.zeros((_M, 13), jnp.float32)], axis=-1)

    w1 = jnp.zeros((16, 64), jnp.float32).at[:6, :].set(W1.T).at[6, :].set(b1)
    w2 = W2.T
    w3 = W3.T
    n = float(_ROWS)

    s1 = _run_stats(_stats1_body, g, nxpad, [w1])
    mean1 = s1[0, :64] / n
    var1 = s1[1, :64] / n - mean1 * mean1
    sc1 = gamma1 / jnp.sqrt(var1 + _EPS)
    w1e = (w1 * sc1[None, :]).at[6, :].add(beta1 - mean1 * sc1)

    s2 = _run_stats(_stats2_body, g, nxpad, [w1e, w2])
    m2 = s2[0, :64] / n
    mean2 = m2 + b2
    var2 = s2[1, :64] / n + 2.0 * b2 * m2 + b2 * b2 - mean2 * mean2
    sc2 = gamma2 / jnp.sqrt(var2 + _EPS)
    w2e = w2 * sc2[None, :]
    shift2 = beta2 + (b2 - mean2) * sc2

    vec2 = jnp.zeros((8, 128), jnp.float32).at[0, :64].set(shift2)
    s3 = _run_stats(_stats3_body, g, nxpad, [w1e, w2e, vec2, w3])
    m3 = s3[0, :] / n
    mean3 = m3 + b3
    var3 = s3[1, :] / n + 2.0 * b3 * m3 + b3 * b3 - mean3 * mean3
    sc3 = gamma3 / jnp.sqrt(var3 + _EPS)
    w3e = w3 * sc3[None, :]
    shift3 = beta3 + (b3 - mean3) * sc3

    vec23 = vec2.at[1, :].set(shift3)
    out = _run_final(g, nxpad, [w1e, w2e, vec23, w3e])

    new_points = out.reshape(_B, _NPOINT, 128).transpose(0, 2, 1)
    return new_xyz.transpose(0, 2, 1), new_points
